# baseline (speedup 1.0000x reference)
import jax
import jax.numpy as jnp
from jax.experimental import pallas as pl
from jax.experimental.pallas import tpu as pltpu

_N = 1024
_BLK = 128
_K = _N // _BLK
_NGRAPH = 16


def _fused_kernel(attn_ref, x_ref, bidx_ref, wagg_ref, bagg_ref,
                  w1_ref, b1_ref, w2_ref, b2_ref,
                  wl1_ref, bl1_ref, lng_ref, lnb_ref, wl2_ref, bl2_ref,
                  out_ref, a_scr, cs_scr):
    i = pl.program_id(0)

    ablk = attn_ref[...]
    acc = ablk[0] * wagg_ref[0, 0]
    for c in range(1, 12):
        acc = acc + ablk[c] * wagg_ref[0, c]
    a = jnp.where(acc + bagg_ref[0, 0] > 0.0, 1.0, 0.0)
    a_scr[pl.ds(i * _BLK, _BLK), :] = a
    part = jnp.sum(a, axis=0, keepdims=True)

    @pl.when(i == 0)
    def _init():
        cs_scr[...] = part

    @pl.when(i > 0)
    def _acc():
        cs_scr[...] = cs_scr[...] + part

    @pl.when(i == _K - 1)
    def _dense():
        a_full = a_scr[...]
        f32 = jnp.float32
        dims = (((0,), (0,)), ((), ()))

        dinv_row = jax.lax.rsqrt(cs_scr[...] + 1.0)
        dinv = jnp.transpose(dinv_row)
        dinv2 = dinv * dinv

        xw = jnp.dot(x_ref[...], w1_ref[...], preferred_element_type=f32)
        z = jax.lax.dot_general(a_full, xw * dinv, dims,
                                preferred_element_type=f32)
        h = z * dinv + xw * dinv2 + b1_ref[...]
        h = jnp.maximum(h, 0.0)

        xw2 = jnp.dot(h, w2_ref[...], preferred_element_type=f32)
        z2 = jax.lax.dot_general(a_full, xw2 * dinv, dims,
                                 preferred_element_type=f32)
        h2 = z2 * dinv + xw2 * dinv2 + b2_ref[...]

        gid = jax.lax.broadcasted_iota(jnp.int32, (_NGRAPH, _N), 0)
        p = (gid == bidx_ref[...]).astype(f32)
        sums = jnp.dot(p, h2, preferred_element_type=f32)
        cnt = jnp.sum(p, axis=1, keepdims=True)
        g = sums / jnp.maximum(cnt, 1.0)

        t = jnp.maximum(jnp.dot(g, wl1_ref[...],
                                preferred_element_type=f32) + bl1_ref[...], 0.0)
        mu = jnp.mean(t, axis=1, keepdims=True)
        var = jnp.mean((t - mu) * (t - mu), axis=1, keepdims=True)
        hn = (t - mu) * jax.lax.rsqrt(var + 1e-5) * lng_ref[...] + lnb_ref[...]
        out_ref[...] = jnp.dot(hn, wl2_ref[...],
                               preferred_element_type=f32) + bl2_ref[...]


def kernel(x, attn_tensor, edge_weight, batch_idx, Wagg, bagg, W1, b1, W2, b2,
           Wl1, bl1, ln_g, ln_b, Wl2, bl2):
    n, in_ch = x.shape
    hid = W1.shape[1]
    out_ch = Wl2.shape[1]

    bidx = batch_idx.astype(jnp.int32).reshape(1, n)
    wagg = Wagg.reshape(1, -1)
    bagg2 = bagg.reshape(1, 1)

    full = lambda shp: pl.BlockSpec(shp, lambda i: (0,) * len(shp))

    return pl.pallas_call(
        _fused_kernel,
        grid=(_K,),
        in_specs=[
            pl.BlockSpec((12, _BLK, _N), lambda i: (0, i, 0)),
            full((n, in_ch)),
            full((1, n)),
            full((1, 12)),
            full((1, 1)),
            full((in_ch, hid)),
            full((1, hid)),
            full((hid, hid)),
            full((1, hid)),
            full((hid, hid)),
            full((1, hid)),
            full((1, hid)),
            full((1, hid)),
            full((hid, out_ch)),
            full((1, out_ch)),
        ],
        out_specs=full((_NGRAPH, out_ch)),
        out_shape=jax.ShapeDtypeStruct((_NGRAPH, out_ch), jnp.float32),
        scratch_shapes=[pltpu.VMEM((_N, _N), jnp.float32),
                        pltpu.VMEM((1, _N), jnp.float32)],
    )(attn_tensor, x, bidx, wagg, bagg2,
      W1, b1.reshape(1, -1), W2, b2.reshape(1, -1),
      Wl1, bl1.reshape(1, -1), ln_g.reshape(1, -1), ln_b.reshape(1, -1),
      Wl2, bl2.reshape(1, -1))


# speedup vs baseline: 1.0197x; 1.0197x over previous
import jax
import jax.numpy as jnp
from jax.experimental import pallas as pl
from jax.experimental.pallas import tpu as pltpu

_N = 1024
_BLK = 256
_K = _N // _BLK
_NGRAPH = 16


def _fused_kernel(attn_ref, x_ref, bidx_ref, wagg_ref, bagg_ref,
                  w1_ref, b1_ref, w2_ref, b2_ref,
                  wl1_ref, bl1_ref, lng_ref, lnb_ref, wl2_ref, bl2_ref,
                  out_ref, a_scr, cs_scr):
    i = pl.program_id(0)

    ablk = attn_ref[...]
    acc = ablk[0] * wagg_ref[0, 0]
    for c in range(1, 12):
        acc = acc + ablk[c] * wagg_ref[0, c]
    a = jnp.where(acc + bagg_ref[0, 0] > 0.0, 1.0, 0.0)
    a_scr[pl.ds(i * _BLK, _BLK), :] = a
    part = jnp.sum(a, axis=0, keepdims=True)

    @pl.when(i == 0)
    def _init():
        cs_scr[...] = part

    @pl.when(i > 0)
    def _acc():
        cs_scr[...] = cs_scr[...] + part

    @pl.when(i == _K - 1)
    def _dense():
        a_full = a_scr[...]
        f32 = jnp.float32
        dims = (((0,), (0,)), ((), ()))

        dinv_row = jax.lax.rsqrt(cs_scr[...] + 1.0)
        dinv = jnp.transpose(dinv_row)
        dinv2 = dinv * dinv

        xw = jnp.dot(x_ref[...], w1_ref[...], preferred_element_type=f32)
        z = jax.lax.dot_general(a_full, xw * dinv, dims,
                                preferred_element_type=f32)
        h = z * dinv + xw * dinv2 + b1_ref[...]
        h = jnp.maximum(h, 0.0)

        xw2 = jnp.dot(h, w2_ref[...], preferred_element_type=f32)
        z2 = jax.lax.dot_general(a_full, xw2 * dinv, dims,
                                 preferred_element_type=f32)
        h2 = z2 * dinv + xw2 * dinv2 + b2_ref[...]

        gid = jax.lax.broadcasted_iota(jnp.int32, (_NGRAPH, _N), 0)
        p = (gid == bidx_ref[...]).astype(f32)
        sums = jnp.dot(p, h2, preferred_element_type=f32)
        cnt = jnp.sum(p, axis=1, keepdims=True)
        g = sums / jnp.maximum(cnt, 1.0)

        t = jnp.maximum(jnp.dot(g, wl1_ref[...],
                                preferred_element_type=f32) + bl1_ref[...], 0.0)
        mu = jnp.mean(t, axis=1, keepdims=True)
        var = jnp.mean((t - mu) * (t - mu), axis=1, keepdims=True)
        hn = (t - mu) * jax.lax.rsqrt(var + 1e-5) * lng_ref[...] + lnb_ref[...]
        out_ref[...] = jnp.dot(hn, wl2_ref[...],
                               preferred_element_type=f32) + bl2_ref[...]


def kernel(x, attn_tensor, edge_weight, batch_idx, Wagg, bagg, W1, b1, W2, b2,
           Wl1, bl1, ln_g, ln_b, Wl2, bl2):
    n, in_ch = x.shape
    hid = W1.shape[1]
    out_ch = Wl2.shape[1]

    bidx = batch_idx.astype(jnp.int32).reshape(1, n)
    wagg = Wagg.reshape(1, -1)
    bagg2 = bagg.reshape(1, 1)

    full = lambda shp: pl.BlockSpec(shp, lambda i: (0,) * len(shp))

    return pl.pallas_call(
        _fused_kernel,
        grid=(_K,),
        in_specs=[
            pl.BlockSpec((12, _BLK, _N), lambda i: (0, i, 0)),
            full((n, in_ch)),
            full((1, n)),
            full((1, 12)),
            full((1, 1)),
            full((in_ch, hid)),
            full((1, hid)),
            full((hid, hid)),
            full((1, hid)),
            full((hid, hid)),
            full((1, hid)),
            full((1, hid)),
            full((1, hid)),
            full((hid, out_ch)),
            full((1, out_ch)),
        ],
        out_specs=full((_NGRAPH, out_ch)),
        out_shape=jax.ShapeDtypeStruct((_NGRAPH, out_ch), jnp.float32),
        scratch_shapes=[pltpu.VMEM((_N, _N), jnp.float32),
                        pltpu.VMEM((1, _N), jnp.float32)],
    )(attn_tensor, x, bidx, wagg, bagg2,
      W1, b1.reshape(1, -1), W2, b2.reshape(1, -1),
      Wl1, bl1.reshape(1, -1), ln_g.reshape(1, -1), ln_b.reshape(1, -1),
      Wl2, bl2.reshape(1, -1))
